# baseline (speedup 1.0000x reference)
import functools

import jax
import jax.numpy as jnp
from jax import lax
from jax.experimental import pallas as pl
from jax.experimental.pallas import tpu as pltpu
from jax.experimental.pallas import tpu_sc as plsc

HID = 768
INNER = 3072
NE = 64
NT = 2048
NA = 2 * NT
EPS = 1e-05
BLK = 64
NB = 128
P = NB * BLK
KTILE = 768
KT = INNER // KTILE
NC = 2
NS = 16
NW = NC * NS
SPLITS = ((0, NB),)


def _gelu(v):
    return 0.5 * v * (1.0 + jnp.tanh(jnp.sqrt(2.0 / jnp.pi) * (v + 0.044715 * v ** 3)))


def _route_body(x_ref, wr_ref, br_ref, rs_ref, xn_ref, d1_ref, d2_ref,
                w1_ref, w2_ref, be_ref, lv_ref, nl_ref):
    xv = x_ref[...]
    logits = jnp.dot(xv, wr_ref[...], preferred_element_type=jnp.float32) + br_ref[...]
    mx = jnp.max(logits, axis=1, keepdims=True)
    ex = jnp.exp(logits - mx)
    rs = ex / jnp.sum(ex, axis=1, keepdims=True)
    rs_ref[...] = rs

    mu = jnp.mean(xv, axis=1, keepdims=True)
    var = jnp.mean((xv - mu) ** 2, axis=1, keepdims=True)
    xn_ref[...] = (xv - mu) / jnp.sqrt(var + EPS)

    lane = lax.broadcasted_iota(jnp.int32, (NT, NE), 1)
    m1 = jnp.max(rs, axis=1, keepdims=True)
    i1 = jnp.min(jnp.where(rs == m1, lane, NE), axis=1, keepdims=True)
    rs2 = jnp.where(lane == i1, -1.0, rs)
    m2 = jnp.max(rs2, axis=1, keepdims=True)
    i2 = jnp.min(jnp.where(rs2 == m2, lane, NE), axis=1, keepdims=True)
    ssum = m1 + m2
    w1_ref[...] = m1 / ssum
    w2_ref[...] = m2 / ssum

    one1 = (lane == i1).astype(jnp.float32)
    one2 = (lane == i2).astype(jnp.float32)

    def excl_cumsum_rows(m):
        c = m
        s = 1
        while s < NT:
            c = c + jnp.concatenate(
                [jnp.zeros((s, NE), jnp.float32), c[:-s, :]], axis=0)
            s *= 2
        return c - m

    c1 = excl_cumsum_rows(one1)
    tot1 = jnp.sum(one1, axis=0, keepdims=True)
    c2 = excl_cumsum_rows(one2) + tot1
    counts = tot1 + jnp.sum(one2, axis=0, keepdims=True)
    nblk = jnp.floor((counts + (BLK - 1)) * (1.0 / BLK))

    def excl_cumsum_lanes(v):
        c = v
        s = 1
        while s < NE:
            c = c + jnp.concatenate(
                [jnp.zeros((1, s), jnp.float32), c[:, :-s]], axis=1)
            s *= 2
        return c - v

    blkoff = excl_cumsum_lanes(nblk)
    poff = blkoff * float(BLK)
    d1_ref[...] = jnp.sum(one1 * (c1 + poff), axis=1, keepdims=True).astype(jnp.int32)
    d2_ref[...] = jnp.sum(one2 * (c2 + poff), axis=1, keepdims=True).astype(jnp.int32)

    bio = lax.broadcasted_iota(jnp.int32, (NB, NE), 0).astype(jnp.float32)
    eio = lax.broadcasted_iota(jnp.int32, (NB, NE), 1)
    be_ref[...] = jnp.max(jnp.where(blkoff <= bio, eio, 0), axis=1, keepdims=True)
    nlive = jnp.sum(nblk, axis=1, keepdims=True)
    lv_ref[...] = (lax.broadcasted_iota(jnp.int32, (NB, 1), 0).astype(jnp.float32)
                   < nlive).astype(jnp.int32)
    nl_ref[...] = jnp.broadcast_to(nlive, (1, 16)).astype(jnp.int32)


def _route(x2, Wr, br):
    f32 = jnp.float32
    i32 = jnp.int32
    return pl.pallas_call(
        _route_body,
        out_shape=[
            jax.ShapeDtypeStruct((NT, NE), f32),
            jax.ShapeDtypeStruct((NT, HID), f32),
            jax.ShapeDtypeStruct((NT, 1), i32),
            jax.ShapeDtypeStruct((NT, 1), i32),
            jax.ShapeDtypeStruct((NT, 1), f32),
            jax.ShapeDtypeStruct((NT, 1), f32),
            jax.ShapeDtypeStruct((NB, 1), i32),
            jax.ShapeDtypeStruct((NB, 1), i32),
            jax.ShapeDtypeStruct((1, 16), i32),
        ],
    )(x2, Wr, br.reshape(1, NE))


def _dispatch_build(dflat, tsnf):
    @functools.partial(
        pl.kernel,
        out_type=jax.ShapeDtypeStruct((P,), jnp.float32),
        mesh=plsc.VectorSubcoreMesh(core_axis_name="c", subcore_axis_name="s"),
        compiler_params=pltpu.CompilerParams(needs_layout_passes=False),
        scratch_types=[pltpu.VMEM((NA,), jnp.int32),
                       pltpu.VMEM((NA,), jnp.float32),
                       pltpu.VMEM((P,), jnp.float32)],
    )
    def k(d_hbm, t_hbm, wv_hbm, d_v, t_v, wv_v):
        wid = lax.axis_index("s") * NC + lax.axis_index("c")

        @pl.when(wid == 0)
        def _():
            pltpu.sync_copy(d_hbm, d_v)
            pltpu.sync_copy(t_hbm, t_v)
            zf = jnp.zeros((16,), jnp.float32)

            def init(i, carry):
                wv_v[pl.ds(i * 16, 16)] = zf
                return carry

            lax.fori_loop(0, P // 16, init, 0)

            def scat(i, carry):
                dd = d_v[pl.ds(i * 16, 16)]
                plsc.store_scatter(wv_v, [dd], t_v[pl.ds(i * 16, 16)])
                return carry

            lax.fori_loop(0, NA // 16, scat, 0)
            pltpu.sync_copy(wv_v, wv_hbm)

    return k(dflat, tsnf)


def _scatter_rows(xn, dd0, dd1):
    tpw = NT // NW

    @functools.partial(
        pl.kernel,
        out_type=jax.ShapeDtypeStruct((P, HID), jnp.float32),
        mesh=plsc.VectorSubcoreMesh(core_axis_name="c", subcore_axis_name="s"),
        compiler_params=pltpu.CompilerParams(needs_layout_passes=False),
        scratch_types=[pltpu.VMEM((tpw,), jnp.int32),
                       pltpu.VMEM((tpw,), jnp.int32),
                       pltpu.VMEM((tpw, HID), jnp.float32),
                       pltpu.SemaphoreType.DMA,
                       pltpu.SemaphoreType.DMA],
    )
    def k(xn_hbm, d0_hbm, d1_hbm, px_hbm, i0_v, i1_v, rows_v, s0, s1):
        wid = lax.axis_index("s") * NC + lax.axis_index("c")
        base = wid * tpw
        pltpu.sync_copy(d0_hbm.at[pl.ds(base, tpw)], i0_v)
        pltpu.sync_copy(d1_hbm.at[pl.ds(base, tpw)], i1_v)
        pltpu.sync_copy(xn_hbm.at[pl.ds(base, tpw)], rows_v)
        c0 = pltpu.make_async_copy(rows_v, px_hbm.at[i0_v], s0)
        c1 = pltpu.make_async_copy(rows_v, px_hbm.at[i1_v], s1)
        c0.start()
        c1.start()
        c0.wait()
        c1.wait()

    return k(xn, dd0, dd1)


def _ffn(be, lv, px, Wfc, Wproj, gamma, beta, bfc, bproj, wvec,
         b_lo, b_n, prev=None):
    IH = INNER // 2
    in_specs = [
        pl.BlockSpec((BLK, HID), lambda b, be, lv: (b, 0)),
        pl.BlockSpec((1, HID, IH), lambda b, be, lv: (be[b + b_lo], 0, 0)),
        pl.BlockSpec((1, HID, IH), lambda b, be, lv: (be[b + b_lo], 0, 1)),
        pl.BlockSpec((1, IH, HID), lambda b, be, lv: (be[b + b_lo], 0, 0)),
        pl.BlockSpec((1, IH, HID), lambda b, be, lv: (be[b + b_lo], 1, 0)),
        pl.BlockSpec((1, 1, HID), lambda b, be, lv: (be[b + b_lo], 0, 0)),
        pl.BlockSpec((1, 1, HID), lambda b, be, lv: (be[b + b_lo], 0, 0)),
        pl.BlockSpec((1, 1, IH), lambda b, be, lv: (be[b + b_lo], 0, 0)),
        pl.BlockSpec((1, 1, IH), lambda b, be, lv: (be[b + b_lo], 0, 1)),
        pl.BlockSpec((1, 1, HID), lambda b, be, lv: (be[b + b_lo], 0, 0)),
        pl.BlockSpec((BLK, 1), lambda b, be, lv: (b + b_lo, 0)),
    ]
    args = [be, lv, px, Wfc, Wfc, Wproj, Wproj, gamma.reshape(NE, 1, HID),
            beta.reshape(NE, 1, HID), bfc.reshape(NE, 1, INNER),
            bfc.reshape(NE, 1, INNER), bproj.reshape(NE, 1, HID), wvec]
    aliases = {}
    if prev is not None:
        in_specs.append(pl.BlockSpec(memory_space=pl.ANY))
        args.append(prev)
        aliases = {13: 0}
    grid_spec = pltpu.PrefetchScalarGridSpec(
        num_scalar_prefetch=2,
        grid=(b_n,),
        in_specs=in_specs,
        out_specs=pl.BlockSpec((BLK, HID), lambda b, be, lv: (b + b_lo, 0)),
    )

    def body(be_ref, lv_ref, x_ref, wfa_ref, wfb_ref, wpa_ref, wpb_ref,
             g_ref, bta_ref, bfa_ref, bfb_ref, bpj_ref, wv_ref, *rest):
        out_ref = rest[-1]
        b = pl.program_id(0)

        @pl.when(lv_ref[b + b_lo] > 0)
        def _():
            cs = x_ref[...] * g_ref[0] + bta_ref[0]
            a1 = jnp.dot(cs, wfa_ref[0], preferred_element_type=jnp.float32) + bfa_ref[0]
            a2 = jnp.dot(cs, wfb_ref[0], preferred_element_type=jnp.float32) + bfb_ref[0]
            o = (jnp.dot(_gelu(a1), wpa_ref[0], preferred_element_type=jnp.float32)
                 + jnp.dot(_gelu(a2), wpb_ref[0], preferred_element_type=jnp.float32))
            out_ref[...] = (o + bpj_ref[0]) * wv_ref[...]

    return pl.pallas_call(
        body,
        grid_spec=grid_spec,
        out_shape=jax.ShapeDtypeStruct((P, HID), jnp.float32),
        input_output_aliases=aliases,
        compiler_params=pltpu.CompilerParams(
            dimension_semantics=("arbitrary",),
            vmem_limit_bytes=100 * 1024 * 1024,
        ),
    )(*args)


def _combine(dd0, dd1, pout):
    tpw = NT // NW

    @functools.partial(
        pl.kernel,
        out_type=jax.ShapeDtypeStruct((NT, HID), jnp.float32),
        mesh=plsc.VectorSubcoreMesh(core_axis_name="c", subcore_axis_name="s"),
        compiler_params=pltpu.CompilerParams(needs_layout_passes=False),
        scratch_types=[pltpu.VMEM((tpw,), jnp.int32),
                       pltpu.VMEM((tpw,), jnp.int32),
                       pltpu.VMEM((tpw, HID), jnp.float32),
                       pltpu.VMEM((tpw, HID), jnp.float32),
                       pltpu.SemaphoreType.DMA,
                       pltpu.SemaphoreType.DMA],
    )
    def k(d0_hbm, d1_hbm, po_hbm, out_hbm, i0_v, i1_v, a_v, b_v, s0, s1):
        wid = lax.axis_index("s") * NC + lax.axis_index("c")
        base = wid * tpw
        pltpu.sync_copy(d0_hbm.at[pl.ds(base, tpw)], i0_v)
        pltpu.sync_copy(d1_hbm.at[pl.ds(base, tpw)], i1_v)
        cp0 = pltpu.async_copy(po_hbm.at[i0_v], a_v, s0)
        cp1 = pltpu.async_copy(po_hbm.at[i1_v], b_v, s1)
        cp0.wait()
        cp1.wait()

        def row(r, carry):
            for c in range(HID // 16):
                sl = pl.ds(c * 16, 16)
                a_v[r, sl] = a_v[r, sl] + b_v[r, sl]
            return carry

        lax.fori_loop(0, tpw, row, 0)
        pltpu.sync_copy(a_v, out_hbm.at[pl.ds(base, tpw)])

    return k(dd0, dd1, pout)


def kernel(x, Wr, br, gamma, beta, Wfc, bfc, Wproj, bproj):
    bsz, q_len, d = x.shape
    x2 = x.reshape(NT, HID)
    rs, xn, d1, d2, w1, w2, be, lv, nl = _route(x2, Wr, br)
    dflat = jnp.concatenate([d1[:, 0], d2[:, 0]])
    tsnf = jnp.concatenate([w1[:, 0], w2[:, 0]])
    wvec = _dispatch_build(dflat, tsnf)
    wv2 = wvec.reshape(P, 1)
    px = _scatter_rows(xn, d1[:, 0], d2[:, 0])
    pout = _ffn(be[:, 0], lv[:, 0], px, Wfc, Wproj, gamma, beta, bfc,
                bproj, wv2, 0, NB)
    out = _combine(d1[:, 0], d2[:, 0], pout)
    return out.reshape(bsz, q_len, d), rs.reshape(bsz, q_len, NE)


# speedup vs baseline: 1.0059x; 1.0059x over previous
import functools

import jax
import jax.numpy as jnp
from jax import lax
from jax.experimental import pallas as pl
from jax.experimental.pallas import tpu as pltpu
from jax.experimental.pallas import tpu_sc as plsc

HID = 768
INNER = 3072
NE = 64
NT = 2048
NA = 2 * NT
EPS = 1e-05
BLK = 64
NB = 128
P = NB * BLK
KTILE = 768
KT = INNER // KTILE
NC = 2
NS = 16
NW = NC * NS
SPLITS = ((0, NB),)


def _gelu(v):
    return 0.5 * v * (1.0 + jnp.tanh(jnp.sqrt(2.0 / jnp.pi) * (v + 0.044715 * v ** 3)))


def _route_body(x_ref, wr_ref, br_ref, rs_ref, xn_ref, d1_ref, d2_ref,
                w1_ref, w2_ref, be_ref, lv_ref, nl_ref):
    xv = x_ref[...]
    logits = jnp.dot(xv, wr_ref[...], preferred_element_type=jnp.float32) + br_ref[...]
    mx = jnp.max(logits, axis=1, keepdims=True)
    ex = jnp.exp(logits - mx)
    rs = ex / jnp.sum(ex, axis=1, keepdims=True)
    rs_ref[...] = rs

    mu = jnp.mean(xv, axis=1, keepdims=True)
    var = jnp.mean((xv - mu) ** 2, axis=1, keepdims=True)
    xn_ref[...] = (xv - mu) / jnp.sqrt(var + EPS)

    lane = lax.broadcasted_iota(jnp.int32, (NT, NE), 1)
    m1 = jnp.max(rs, axis=1, keepdims=True)
    i1 = jnp.min(jnp.where(rs == m1, lane, NE), axis=1, keepdims=True)
    rs2 = jnp.where(lane == i1, -1.0, rs)
    m2 = jnp.max(rs2, axis=1, keepdims=True)
    i2 = jnp.min(jnp.where(rs2 == m2, lane, NE), axis=1, keepdims=True)
    ssum = m1 + m2
    w1_ref[...] = m1 / ssum
    w2_ref[...] = m2 / ssum

    one1 = (lane == i1).astype(jnp.float32)
    one2 = (lane == i2).astype(jnp.float32)

    def excl_cumsum_rows(m):
        c = m
        s = 1
        while s < NT:
            c = c + jnp.concatenate(
                [jnp.zeros((s, NE), jnp.float32), c[:-s, :]], axis=0)
            s *= 2
        return c - m

    c1 = excl_cumsum_rows(one1)
    tot1 = jnp.sum(one1, axis=0, keepdims=True)
    c2 = excl_cumsum_rows(one2) + tot1
    counts = tot1 + jnp.sum(one2, axis=0, keepdims=True)
    nblk = jnp.floor((counts + (BLK - 1)) * (1.0 / BLK))

    def excl_cumsum_lanes(v):
        c = v
        s = 1
        while s < NE:
            c = c + jnp.concatenate(
                [jnp.zeros((1, s), jnp.float32), c[:, :-s]], axis=1)
            s *= 2
        return c - v

    blkoff = excl_cumsum_lanes(nblk)
    poff = blkoff * float(BLK)
    d1_ref[...] = jnp.sum(one1 * (c1 + poff), axis=1, keepdims=True).astype(jnp.int32)
    d2_ref[...] = jnp.sum(one2 * (c2 + poff), axis=1, keepdims=True).astype(jnp.int32)

    bio = lax.broadcasted_iota(jnp.int32, (NB, NE), 0).astype(jnp.float32)
    eio = lax.broadcasted_iota(jnp.int32, (NB, NE), 1)
    be_ref[...] = jnp.max(jnp.where(blkoff <= bio, eio, 0), axis=1, keepdims=True)
    nlive = jnp.sum(nblk, axis=1, keepdims=True)
    lv_ref[...] = (lax.broadcasted_iota(jnp.int32, (NB, 1), 0).astype(jnp.float32)
                   < nlive).astype(jnp.int32)
    nl_ref[...] = jnp.broadcast_to(nlive, (1, 16)).astype(jnp.int32)


def _route(x2, Wr, br):
    f32 = jnp.float32
    i32 = jnp.int32
    return pl.pallas_call(
        _route_body,
        out_shape=[
            jax.ShapeDtypeStruct((NT, NE), f32),
            jax.ShapeDtypeStruct((NT, HID), f32),
            jax.ShapeDtypeStruct((NT, 1), i32),
            jax.ShapeDtypeStruct((NT, 1), i32),
            jax.ShapeDtypeStruct((NT, 1), f32),
            jax.ShapeDtypeStruct((NT, 1), f32),
            jax.ShapeDtypeStruct((NB, 1), i32),
            jax.ShapeDtypeStruct((NB, 1), i32),
            jax.ShapeDtypeStruct((1, 16), i32),
        ],
    )(x2, Wr, br.reshape(1, NE))


def _dispatch_build(dflat, tsnf):
    @functools.partial(
        pl.kernel,
        out_type=jax.ShapeDtypeStruct((P,), jnp.float32),
        mesh=plsc.VectorSubcoreMesh(core_axis_name="c", subcore_axis_name="s"),
        compiler_params=pltpu.CompilerParams(needs_layout_passes=False),
        scratch_types=[pltpu.VMEM((NA,), jnp.int32),
                       pltpu.VMEM((NA,), jnp.float32),
                       pltpu.VMEM((P,), jnp.float32)],
    )
    def k(d_hbm, t_hbm, wv_hbm, d_v, t_v, wv_v):
        wid = lax.axis_index("s") * NC + lax.axis_index("c")

        @pl.when(wid == 0)
        def _():
            pltpu.sync_copy(d_hbm, d_v)
            pltpu.sync_copy(t_hbm, t_v)
            zf = jnp.zeros((16,), jnp.float32)

            def init(i, carry):
                wv_v[pl.ds(i * 16, 16)] = zf
                return carry

            lax.fori_loop(0, P // 16, init, 0)

            def scat(i, carry):
                dd = d_v[pl.ds(i * 16, 16)]
                plsc.store_scatter(wv_v, [dd], t_v[pl.ds(i * 16, 16)])
                return carry

            lax.fori_loop(0, NA // 16, scat, 0)
            pltpu.sync_copy(wv_v, wv_hbm)

    return k(dflat, tsnf)


def _scatter_rows(xn, dd0, dd1):
    tpw = NT // NW

    @functools.partial(
        pl.kernel,
        out_type=jax.ShapeDtypeStruct((P, HID), jnp.float32),
        mesh=plsc.VectorSubcoreMesh(core_axis_name="c", subcore_axis_name="s"),
        compiler_params=pltpu.CompilerParams(needs_layout_passes=False),
        scratch_types=[pltpu.VMEM((tpw,), jnp.int32),
                       pltpu.VMEM((tpw,), jnp.int32),
                       pltpu.VMEM((tpw, HID), jnp.float32),
                       pltpu.SemaphoreType.DMA,
                       pltpu.SemaphoreType.DMA],
    )
    def k(xn_hbm, d0_hbm, d1_hbm, px_hbm, i0_v, i1_v, rows_v, s0, s1):
        wid = lax.axis_index("s") * NC + lax.axis_index("c")
        base = wid * tpw
        pltpu.sync_copy(d0_hbm.at[pl.ds(base, tpw)], i0_v)
        pltpu.sync_copy(d1_hbm.at[pl.ds(base, tpw)], i1_v)
        pltpu.sync_copy(xn_hbm.at[pl.ds(base, tpw)], rows_v)
        c0 = pltpu.make_async_copy(rows_v, px_hbm.at[i0_v], s0)
        c1 = pltpu.make_async_copy(rows_v, px_hbm.at[i1_v], s1)
        c0.start()
        c1.start()
        c0.wait()
        c1.wait()

    return k(xn, dd0, dd1)


def _ffn(be, lv, px, Wfc, Wproj, gamma, beta, bfc, bproj, wvec,
         b_lo, b_n, prev=None):
    in_specs = [
        pl.BlockSpec((BLK, HID), lambda b, be, lv: (b, 0)),
        pl.BlockSpec((1, HID, INNER), lambda b, be, lv: (be[b + b_lo], 0, 0)),
        pl.BlockSpec((1, INNER, HID), lambda b, be, lv: (be[b + b_lo], 0, 0)),
        pl.BlockSpec((1, 1, HID), lambda b, be, lv: (be[b + b_lo], 0, 0)),
        pl.BlockSpec((1, 1, HID), lambda b, be, lv: (be[b + b_lo], 0, 0)),
        pl.BlockSpec((1, 1, INNER), lambda b, be, lv: (be[b + b_lo], 0, 0)),
        pl.BlockSpec((1, 1, HID), lambda b, be, lv: (be[b + b_lo], 0, 0)),
        pl.BlockSpec((BLK, 1), lambda b, be, lv: (b + b_lo, 0)),
    ]
    args = [be, lv, px, Wfc, Wproj, gamma.reshape(NE, 1, HID),
            beta.reshape(NE, 1, HID), bfc.reshape(NE, 1, INNER),
            bproj.reshape(NE, 1, HID), wvec]
    aliases = {}
    if prev is not None:
        in_specs.append(pl.BlockSpec(memory_space=pl.ANY))
        args.append(prev)
        aliases = {10: 0}
    grid_spec = pltpu.PrefetchScalarGridSpec(
        num_scalar_prefetch=2,
        grid=(b_n,),
        in_specs=in_specs,
        out_specs=pl.BlockSpec((BLK, HID), lambda b, be, lv: (b + b_lo, 0)),
    )

    def body(be_ref, lv_ref, x_ref, wfc_ref, wpj_ref, g_ref, bta_ref,
             bfc_ref, bpj_ref, wv_ref, *rest):
        out_ref = rest[-1]
        b = pl.program_id(0)

        @pl.when(lv_ref[b + b_lo] > 0)
        def _():
            cs = x_ref[...] * g_ref[0] + bta_ref[0]
            a = jnp.dot(cs, wfc_ref[0], preferred_element_type=jnp.float32) + bfc_ref[0]
            a = _gelu(a)
            o = jnp.dot(a, wpj_ref[0], preferred_element_type=jnp.float32)
            out_ref[...] = (o + bpj_ref[0]) * wv_ref[...]

    return pl.pallas_call(
        body,
        grid_spec=grid_spec,
        out_shape=jax.ShapeDtypeStruct((P, HID), jnp.float32),
        input_output_aliases=aliases,
        compiler_params=pltpu.CompilerParams(
            dimension_semantics=("arbitrary",),
            vmem_limit_bytes=100 * 1024 * 1024,
        ),
    )(*args)


def _combine(dd0, dd1, pout):
    tpw = NT // NW

    @functools.partial(
        pl.kernel,
        out_type=jax.ShapeDtypeStruct((NT, HID), jnp.float32),
        mesh=plsc.VectorSubcoreMesh(core_axis_name="c", subcore_axis_name="s"),
        compiler_params=pltpu.CompilerParams(needs_layout_passes=False),
        scratch_types=[pltpu.VMEM((tpw,), jnp.int32),
                       pltpu.VMEM((tpw,), jnp.int32),
                       pltpu.VMEM((tpw, HID), jnp.float32),
                       pltpu.VMEM((tpw, HID), jnp.float32),
                       pltpu.SemaphoreType.DMA,
                       pltpu.SemaphoreType.DMA],
    )
    def k(d0_hbm, d1_hbm, po_hbm, out_hbm, i0_v, i1_v, a_v, b_v, s0, s1):
        wid = lax.axis_index("s") * NC + lax.axis_index("c")
        base = wid * tpw
        pltpu.sync_copy(d0_hbm.at[pl.ds(base, tpw)], i0_v)
        pltpu.sync_copy(d1_hbm.at[pl.ds(base, tpw)], i1_v)
        cp0 = pltpu.async_copy(po_hbm.at[i0_v], a_v, s0)
        cp1 = pltpu.async_copy(po_hbm.at[i1_v], b_v, s1)
        cp0.wait()
        cp1.wait()

        def row(r, carry):
            for c in range(HID // 16):
                sl = pl.ds(c * 16, 16)
                a_v[r, sl] = a_v[r, sl] + b_v[r, sl]
            return carry

        lax.fori_loop(0, tpw, row, 0)
        pltpu.sync_copy(a_v, out_hbm.at[pl.ds(base, tpw)])

    return k(dd0, dd1, pout)


def kernel(x, Wr, br, gamma, beta, Wfc, bfc, Wproj, bproj):
    bsz, q_len, d = x.shape
    x2 = x.reshape(NT, HID)
    rs, xn, d1, d2, w1, w2, be, lv, nl = _route(x2, Wr, br)
    dflat = jnp.concatenate([d1[:, 0], d2[:, 0]])
    tsnf = jnp.concatenate([w1[:, 0], w2[:, 0]])
    wvec = _dispatch_build(dflat, tsnf)
    wv2 = wvec.reshape(P, 1)
    px = _scatter_rows(xn, d1[:, 0], d2[:, 0])
    pout = _ffn(be[:, 0], lv[:, 0], px, Wfc, Wproj, gamma, beta, bfc,
                bproj, wv2, 0, NB)
    out = _combine(d1[:, 0], d2[:, 0], pout)
    return out.reshape(bsz, q_len, d), rs.reshape(bsz, q_len, NE)
